# Initial kernel scaffold; baseline (speedup 1.0000x reference)
import jax
import jax.numpy as jnp
from jax.experimental import pallas as pl


def kernel(context, anchors, W, b):
    raise NotImplementedError("write your pallas kernel here")


# speedup vs baseline: 8.3089x; 8.3089x over previous
import jax
import jax.numpy as jnp
from jax.experimental import pallas as pl
from jax.experimental.pallas import tpu as pltpu

N_CTX = 8192
N_ANCHOR = 1024
D_IN = 1024
TOPK = 64

BLK = 256
VAL_ITERS = 31
IDX_ITERS = 10


def _agl_kernel(ctx_ref, anc_ref, w_ref, b_ref, out_ref, cn_ref, an_ref):
    f32 = jnp.float32
    bf16 = jnp.bfloat16
    i32 = jnp.int32

    @pl.when(pl.program_id(0) == 0)
    def _():
        g = jax.lax.dot_general(
            w_ref[...].astype(bf16), anc_ref[...].astype(bf16),
            (((0,), (1,)), ((), ())), preferred_element_type=f32)
        a0 = jnp.maximum(g[0:1, :] + b_ref[0, 0], 0.0)
        a1 = jnp.maximum(g[1:2, :] + b_ref[0, 1], 0.0)
        an_ref[0:1, :] = a0.astype(bf16).astype(f32)
        an_ref[1:2, :] = a1.astype(bf16).astype(f32)

    cn = jnp.maximum(
        jax.lax.dot_general(
            ctx_ref[...].astype(bf16), w_ref[...].astype(bf16),
            (((1,), (0,)), ((), ())), preferred_element_type=f32)
        + b_ref[...], 0.0)
    cn_ref[...] = cn

    c0 = cn[:, 0:1].astype(bf16).astype(f32)
    c1 = cn[:, 1:2].astype(bf16).astype(f32)
    att = c0 * an_ref[0:1, :] + c1 * an_ref[1:2, :]

    keys = jax.lax.bitcast_convert_type(att, i32)

    lo = jnp.zeros((BLK, 1), i32)
    hi = jnp.max(keys, axis=1, keepdims=True) + 1

    def vbody(_, lohi):
        lo, hi = lohi
        mid = lo + jax.lax.shift_right_logical(hi - lo, 1)
        cnt = jnp.sum((keys >= mid).astype(i32), axis=1, keepdims=True)
        ge = cnt >= TOPK
        return jnp.where(ge, mid, lo), jnp.where(ge, hi, mid)

    lo, _ = jax.lax.fori_loop(0, VAL_ITERS, vbody, (lo, hi))

    gt = keys > lo
    eq = keys == lo
    need = TOPK - jnp.sum(gt.astype(i32), axis=1, keepdims=True)
    col = jax.lax.broadcasted_iota(i32, (BLK, N_ANCHOR), 1)

    lo_j = jnp.full((BLK, 1), -1, i32)
    hi_j = jnp.full((BLK, 1), N_ANCHOR - 1, i32)

    def jbody(_, lohi):
        lo_j, hi_j = lohi
        mid = lo_j + jax.lax.shift_right_logical(hi_j - lo_j, 1)
        cnt = jnp.sum((eq & (col <= mid)).astype(i32), axis=1, keepdims=True)
        ge = cnt >= need
        return jnp.where(ge, lo_j, mid), jnp.where(ge, mid, hi_j)

    _, hi_j = jax.lax.fori_loop(0, IDX_ITERS, jbody, (lo_j, hi_j))

    mask = gt | (eq & (col <= hi_j))
    out_ref[...] = jnp.where(mask, att, 0.0)


@jax.jit
def kernel(context, anchors, W, b):
    b2 = b.reshape(1, 2)
    grid = N_CTX // BLK
    out, cn = pl.pallas_call(
        _agl_kernel,
        grid=(grid,),
        in_specs=[
            pl.BlockSpec((BLK, D_IN), lambda i: (i, 0)),
            pl.BlockSpec((N_ANCHOR, D_IN), lambda i: (0, 0)),
            pl.BlockSpec((D_IN, 2), lambda i: (0, 0)),
            pl.BlockSpec((1, 2), lambda i: (0, 0)),
        ],
        out_specs=[
            pl.BlockSpec((BLK, N_ANCHOR), lambda i: (i, 0)),
            pl.BlockSpec((BLK, 2), lambda i: (i, 0)),
        ],
        out_shape=[
            jax.ShapeDtypeStruct((N_CTX, N_ANCHOR), jnp.float32),
            jax.ShapeDtypeStruct((N_CTX, 2), jnp.float32),
        ],
        scratch_shapes=[pltpu.VMEM((2, N_ANCHOR), jnp.float32)],
    )(context, anchors, W, b2)
    return out, cn


# speedup vs baseline: 9.0021x; 1.0834x over previous
import jax
import jax.numpy as jnp
from jax.experimental import pallas as pl
from jax.experimental.pallas import tpu as pltpu

N_CTX = 8192
N_ANCHOR = 1024
D_IN = 1024
TOPK = 64

BLK = 256
N_BINS = 256
VAL_ITERS = 20
IDX_ITERS = 10
PREP_ITERS = 31


def _kth_largest_key(mat_keys, k, iters):
    i32 = jnp.int32
    rows = mat_keys.shape[0]
    lo = jnp.zeros((rows, 1), i32)
    hi = jnp.max(mat_keys, axis=1, keepdims=True) + 1

    def body(_, lohi):
        lo, hi = lohi
        mid = lo + jax.lax.shift_right_logical(hi - lo, 1)
        cnt = jnp.sum((mat_keys >= mid).astype(i32), axis=1, keepdims=True)
        ge = cnt >= k
        return jnp.where(ge, mid, lo), jnp.where(ge, hi, mid)

    lo, _ = jax.lax.fori_loop(0, iters, body, (lo, hi))
    return lo


def _agl_kernel(ctx_ref, anc_ref, w_ref, b_ref, out_ref, cn_ref, an_ref,
                tab_ref):
    f32 = jnp.float32
    bf16 = jnp.bfloat16
    i32 = jnp.int32

    @pl.when(pl.program_id(0) == 0)
    def _():
        g = jax.lax.dot_general(
            w_ref[...].astype(bf16), anc_ref[...].astype(bf16),
            (((0,), (1,)), ((), ())), preferred_element_type=f32)
        a0 = jnp.maximum(g[0:1, :] + b_ref[0, 0], 0.0)
        a1 = jnp.maximum(g[1:2, :] + b_ref[0, 1], 0.0)
        a0 = a0.astype(bf16).astype(f32)
        a1 = a1.astype(bf16).astype(f32)
        an_ref[0:1, :] = a0
        an_ref[1:2, :] = a1

        d = jax.lax.broadcasted_iota(i32, (N_BINS, 1), 0).astype(f32)
        ta = d * (1.0 / N_BINS)
        tb = (d + 1.0) * (1.0 / N_BINS)
        ga = (1.0 - ta) * a0 + ta * a1
        gb = (1.0 - tb) * a0 + tb * a1
        lo_d = jnp.minimum(ga, gb)
        hi_d = jnp.maximum(ga, gb)
        klo = _kth_largest_key(
            jax.lax.bitcast_convert_type(lo_d, i32), TOPK, PREP_ITERS)
        khi = _kth_largest_key(
            jax.lax.bitcast_convert_type(hi_d, i32), TOPK, PREP_ITERS)
        t_lo = jax.lax.bitcast_convert_type(klo, f32) * (1.0 - 6e-3)
        t_hi = jax.lax.bitcast_convert_type(khi, f32) * (1.0 + 6e-3)
        tab_ref[...] = jnp.concatenate([t_lo, t_hi], axis=1)

    cn = jnp.maximum(
        jax.lax.dot_general(
            ctx_ref[...].astype(bf16), w_ref[...].astype(bf16),
            (((1,), (0,)), ((), ())), preferred_element_type=f32)
        + b_ref[...], 0.0)
    cn_ref[...] = cn

    c0 = cn[:, 0:1].astype(bf16).astype(f32)
    c1 = cn[:, 1:2].astype(bf16).astype(f32)
    att = c0 * an_ref[0:1, :] + c1 * an_ref[1:2, :]

    keys = jax.lax.bitcast_convert_type(att, i32)

    s = c0 + c1
    tau = jnp.where(s > 0.0, c1 / s, 0.0)
    bin_f = jnp.clip(jnp.floor(tau * N_BINS), 0.0, N_BINS - 1.0)
    bin_i = bin_f.astype(i32)
    bins = jax.lax.broadcasted_iota(i32, (BLK, N_BINS), 1)
    onehot = jnp.where(bins == bin_i, 1.0, 0.0).astype(f32)
    t2 = jax.lax.dot_general(
        onehot, tab_ref[...], (((1,), (0,)), ((), ())),
        preferred_element_type=f32)
    lo_f = jnp.maximum(s * t2[:, 0:1], 0.0)
    hi_f = s * t2[:, 1:2]
    lo = jax.lax.bitcast_convert_type(lo_f, i32)
    hi = jax.lax.bitcast_convert_type(hi_f, i32) + 1
    hi = jnp.maximum(hi, lo + 1)

    def vbody(_, lohi):
        lo, hi = lohi
        mid = lo + jax.lax.shift_right_logical(hi - lo, 1)
        cnt = jnp.sum((keys >= mid).astype(i32), axis=1, keepdims=True)
        ge = cnt >= TOPK
        return jnp.where(ge, mid, lo), jnp.where(ge, hi, mid)

    lo, _ = jax.lax.fori_loop(0, VAL_ITERS, vbody, (lo, hi))

    gt = keys > lo
    eq = keys == lo
    need = TOPK - jnp.sum(gt.astype(i32), axis=1, keepdims=True)
    col = jax.lax.broadcasted_iota(i32, (BLK, N_ANCHOR), 1)

    lo_j = jnp.full((BLK, 1), -1, i32)
    hi_j = jnp.full((BLK, 1), N_ANCHOR - 1, i32)

    def jbody(_, lohi):
        lo_j, hi_j = lohi
        mid = lo_j + jax.lax.shift_right_logical(hi_j - lo_j, 1)
        cnt = jnp.sum((eq & (col <= mid)).astype(i32), axis=1, keepdims=True)
        ge = cnt >= need
        return jnp.where(ge, lo_j, mid), jnp.where(ge, mid, hi_j)

    _, hi_j = jax.lax.fori_loop(0, IDX_ITERS, jbody, (lo_j, hi_j))

    mask = gt | (eq & (col <= hi_j))
    out_ref[...] = jnp.where(mask, att, 0.0)


@jax.jit
def kernel(context, anchors, W, b):
    b2 = b.reshape(1, 2)
    grid = N_CTX // BLK
    out, cn = pl.pallas_call(
        _agl_kernel,
        grid=(grid,),
        in_specs=[
            pl.BlockSpec((BLK, D_IN), lambda i: (i, 0)),
            pl.BlockSpec((N_ANCHOR, D_IN), lambda i: (0, 0)),
            pl.BlockSpec((D_IN, 2), lambda i: (0, 0)),
            pl.BlockSpec((1, 2), lambda i: (0, 0)),
        ],
        out_specs=[
            pl.BlockSpec((BLK, N_ANCHOR), lambda i: (i, 0)),
            pl.BlockSpec((BLK, 2), lambda i: (i, 0)),
        ],
        out_shape=[
            jax.ShapeDtypeStruct((N_CTX, N_ANCHOR), jnp.float32),
            jax.ShapeDtypeStruct((N_CTX, 2), jnp.float32),
        ],
        scratch_shapes=[pltpu.VMEM((2, N_ANCHOR), jnp.float32),
                        pltpu.VMEM((N_BINS, 2), jnp.float32)],
    )(context, anchors, W, b2)
    return out, cn


# speedup vs baseline: 9.4772x; 1.0528x over previous
import jax
import jax.numpy as jnp
from jax.experimental import pallas as pl
from jax.experimental.pallas import tpu as pltpu

N_CTX = 8192
N_ANCHOR = 1024
D_IN = 1024
TOPK = 64

BLK = 256
N_BINS = 512
VAL_ITERS = 16
IDX_ITERS = 10
PREP_ITERS = 31


def _kth_largest_key(mat_keys, k, iters):
    i32 = jnp.int32
    rows = mat_keys.shape[0]
    lo = jnp.zeros((rows, 1), i32)
    hi = jnp.max(mat_keys, axis=1, keepdims=True) + 1

    def body(_, lohi):
        lo, hi = lohi
        mid = lo + jax.lax.shift_right_logical(hi - lo, 1)
        cnt = jnp.sum((mat_keys >= mid).astype(i32), axis=1, keepdims=True)
        ge = cnt >= k
        return jnp.where(ge, mid, lo), jnp.where(ge, hi, mid)

    lo, _ = jax.lax.fori_loop(0, iters, body, (lo, hi))
    return lo


def _agl_kernel(ctx_ref, anc_ref, w_ref, b_ref, out_ref, cn_ref, an_ref,
                tab_ref):
    f32 = jnp.float32
    bf16 = jnp.bfloat16
    i32 = jnp.int32

    @pl.when(pl.program_id(0) == 0)
    def _():
        g = jax.lax.dot_general(
            w_ref[...].astype(bf16), anc_ref[...].astype(bf16),
            (((0,), (1,)), ((), ())), preferred_element_type=f32)
        a0 = jnp.maximum(g[0:1, :] + b_ref[0, 0], 0.0)
        a1 = jnp.maximum(g[1:2, :] + b_ref[0, 1], 0.0)
        a0 = a0.astype(bf16).astype(f32)
        a1 = a1.astype(bf16).astype(f32)
        an_ref[0:1, :] = a0
        an_ref[1:2, :] = a1

        d = jax.lax.broadcasted_iota(i32, (N_BINS, 1), 0).astype(f32)
        ta = d * (1.0 / N_BINS)
        tb = (d + 1.0) * (1.0 / N_BINS)
        ga = (1.0 - ta) * a0 + ta * a1
        gb = (1.0 - tb) * a0 + tb * a1
        lo_d = jnp.minimum(ga, gb)
        hi_d = jnp.maximum(ga, gb)
        klo = _kth_largest_key(
            jax.lax.bitcast_convert_type(lo_d, i32), TOPK, PREP_ITERS)
        khi = _kth_largest_key(
            jax.lax.bitcast_convert_type(hi_d, i32), TOPK, PREP_ITERS)
        t_lo = jax.lax.bitcast_convert_type(klo, f32) * (1.0 - 2e-4)
        t_hi = jax.lax.bitcast_convert_type(khi, f32) * (1.0 + 2e-4)
        tab_ref[...] = jnp.concatenate([t_lo, t_hi], axis=1)

    cn = jnp.maximum(
        jax.lax.dot_general(
            ctx_ref[...].astype(bf16), w_ref[...].astype(bf16),
            (((1,), (0,)), ((), ())), preferred_element_type=f32)
        + b_ref[...], 0.0)
    cn_ref[...] = cn

    c0 = cn[:, 0:1].astype(bf16).astype(f32)
    c1 = cn[:, 1:2].astype(bf16).astype(f32)
    att = c0 * an_ref[0:1, :] + c1 * an_ref[1:2, :]

    keys = jax.lax.bitcast_convert_type(att, i32)

    s = c0 + c1
    tau = jnp.where(s > 0.0, c1 / s, 0.0)
    bin_f = jnp.clip(jnp.floor(tau * N_BINS), 0.0, N_BINS - 1.0)
    bin_i = bin_f.astype(i32)
    bins = jax.lax.broadcasted_iota(i32, (BLK, N_BINS), 1)
    onehot = jnp.where(bins == bin_i, 1.0, 0.0).astype(f32)
    t2 = jax.lax.dot_general(
        onehot, tab_ref[...], (((1,), (0,)), ((), ())),
        preferred_element_type=f32,
        precision=jax.lax.Precision.HIGHEST)
    lo_f = jnp.maximum(s * t2[:, 0:1], 0.0)
    hi_f = s * t2[:, 1:2]
    lo = jax.lax.bitcast_convert_type(lo_f, i32)
    hi = jax.lax.bitcast_convert_type(hi_f, i32) + 1
    hi = jnp.maximum(hi, lo + 1)

    def vbody(_, lohi):
        lo, hi = lohi
        mid = lo + jax.lax.shift_right_logical(hi - lo, 1)
        cnt = jnp.sum((keys >= mid).astype(i32), axis=1, keepdims=True)
        ge = cnt >= TOPK
        return jnp.where(ge, mid, lo), jnp.where(ge, hi, mid)

    lo, _ = jax.lax.fori_loop(0, VAL_ITERS, vbody, (lo, hi))

    gt = keys > lo
    eq = keys == lo
    need = TOPK - jnp.sum(gt.astype(i32), axis=1, keepdims=True)
    col = jax.lax.broadcasted_iota(i32, (BLK, N_ANCHOR), 1)
    e = jnp.where(eq, col, N_ANCHOR * 2)

    lo_j = jnp.full((BLK, 1), -1, i32)
    hi_j = jnp.full((BLK, 1), N_ANCHOR - 1, i32)

    def jbody(_, lohi):
        lo_j, hi_j = lohi
        mid = lo_j + jax.lax.shift_right_logical(hi_j - lo_j, 1)
        cnt = jnp.sum((e <= mid).astype(i32), axis=1, keepdims=True)
        ge = cnt >= need
        return jnp.where(ge, lo_j, mid), jnp.where(ge, mid, hi_j)

    _, hi_j = jax.lax.fori_loop(0, IDX_ITERS, jbody, (lo_j, hi_j))

    mask = gt | (e <= hi_j)
    out_ref[...] = jnp.where(mask, att, 0.0)


@jax.jit
def kernel(context, anchors, W, b):
    b2 = b.reshape(1, 2)
    grid = N_CTX // BLK
    out, cn = pl.pallas_call(
        _agl_kernel,
        grid=(grid,),
        in_specs=[
            pl.BlockSpec((BLK, D_IN), lambda i: (i, 0)),
            pl.BlockSpec((N_ANCHOR, D_IN), lambda i: (0, 0)),
            pl.BlockSpec((D_IN, 2), lambda i: (0, 0)),
            pl.BlockSpec((1, 2), lambda i: (0, 0)),
        ],
        out_specs=[
            pl.BlockSpec((BLK, N_ANCHOR), lambda i: (i, 0)),
            pl.BlockSpec((BLK, 2), lambda i: (i, 0)),
        ],
        out_shape=[
            jax.ShapeDtypeStruct((N_CTX, N_ANCHOR), jnp.float32),
            jax.ShapeDtypeStruct((N_CTX, 2), jnp.float32),
        ],
        scratch_shapes=[pltpu.VMEM((2, N_ANCHOR), jnp.float32),
                        pltpu.VMEM((N_BINS, 2), jnp.float32)],
    )(context, anchors, W, b2)
    return out, cn


# speedup vs baseline: 13.4902x; 1.4234x over previous
import jax
import jax.numpy as jnp
from jax.experimental import pallas as pl
from jax.experimental.pallas import tpu as pltpu

N_CTX = 8192
N_ANCHOR = 1024
D_IN = 1024
TOPK = 64

BLK = 512
N_BINS = 512
VAL_ITERS = 16
IDX_ITERS = 10
PREP_ITERS = 31


def _kth_largest_key(mat_keys, k, iters):
    i32 = jnp.int32
    rows = mat_keys.shape[0]
    lo = jnp.zeros((rows, 1), i32)
    hi = jnp.max(mat_keys, axis=1, keepdims=True) + 1

    def body(_, lohi):
        lo, hi = lohi
        mid = lo + jax.lax.shift_right_logical(hi - lo, 1)
        cnt = jnp.sum((mat_keys >= mid).astype(i32), axis=1, keepdims=True)
        ge = cnt >= k
        return jnp.where(ge, mid, lo), jnp.where(ge, hi, mid)

    lo, _ = jax.lax.fori_loop(0, iters, body, (lo, hi))
    return lo


def _agl_kernel(ctx_ref, anc_ref, w_ref, b_ref, out_ref, cn_ref, an_ref,
                tab_ref):
    f32 = jnp.float32
    bf16 = jnp.bfloat16
    i32 = jnp.int32

    @pl.when(pl.program_id(0) == 0)
    def _():
        g = jax.lax.dot_general(
            w_ref[...].astype(bf16), anc_ref[...].astype(bf16),
            (((0,), (1,)), ((), ())), preferred_element_type=f32)
        a0 = jnp.maximum(g[0:1, :] + b_ref[0, 0], 0.0)
        a1 = jnp.maximum(g[1:2, :] + b_ref[0, 1], 0.0)
        a0 = a0.astype(bf16).astype(f32)
        a1 = a1.astype(bf16).astype(f32)
        an_ref[0:1, :] = a0
        an_ref[1:2, :] = a1

        d = jax.lax.broadcasted_iota(i32, (N_BINS, 1), 0).astype(f32)
        ta = d * (1.0 / N_BINS)
        tb = (d + 1.0) * (1.0 / N_BINS)
        ga = (1.0 - ta) * a0 + ta * a1
        gb = (1.0 - tb) * a0 + tb * a1
        lo_d = jnp.minimum(ga, gb)
        hi_d = jnp.maximum(ga, gb)
        klo = _kth_largest_key(
            jax.lax.bitcast_convert_type(lo_d, i32), TOPK, PREP_ITERS)
        khi = _kth_largest_key(
            jax.lax.bitcast_convert_type(hi_d, i32), TOPK, PREP_ITERS)
        t_lo = jax.lax.bitcast_convert_type(klo, f32) * (1.0 - 2e-4)
        t_hi = jax.lax.bitcast_convert_type(khi, f32) * (1.0 + 2e-4)
        tab_ref[...] = jnp.concatenate([t_lo, t_hi], axis=1)

    cn = jnp.maximum(
        jax.lax.dot_general(
            ctx_ref[...].astype(bf16), w_ref[...].astype(bf16),
            (((1,), (0,)), ((), ())), preferred_element_type=f32)
        + b_ref[...], 0.0)
    cn_ref[...] = cn

    c0 = cn[:, 0:1].astype(bf16).astype(f32)
    c1 = cn[:, 1:2].astype(bf16).astype(f32)
    att = c0 * an_ref[0:1, :] + c1 * an_ref[1:2, :]

    keys = jax.lax.bitcast_convert_type(att, i32)

    s = c0 + c1
    tau = jnp.where(s > 0.0, c1 / s, 0.0)
    bin_f = jnp.clip(jnp.floor(tau * N_BINS), 0.0, N_BINS - 1.0)
    bin_i = bin_f.astype(i32)
    bins = jax.lax.broadcasted_iota(i32, (BLK, N_BINS), 1)
    onehot = jnp.where(bins == bin_i, 1.0, 0.0).astype(f32)
    t2 = jax.lax.dot_general(
        onehot, tab_ref[...], (((1,), (0,)), ((), ())),
        preferred_element_type=f32,
        precision=jax.lax.Precision.HIGHEST)
    lo_f = jnp.maximum(s * t2[:, 0:1], 0.0)
    hi_f = s * t2[:, 1:2]
    lo = jax.lax.bitcast_convert_type(lo_f, i32)
    hi = jax.lax.bitcast_convert_type(hi_f, i32) + 1
    hi = jnp.maximum(hi, lo + 1)

    def vbody(_, lohi):
        lo, hi = lohi
        mid = lo + jax.lax.shift_right_logical(hi - lo, 1)
        cnt = jnp.sum((keys >= mid).astype(i32), axis=1, keepdims=True)
        ge = cnt >= TOPK
        return jnp.where(ge, mid, lo), jnp.where(ge, hi, mid)

    lo, _ = jax.lax.fori_loop(0, VAL_ITERS, vbody, (lo, hi), unroll=4)

    gt = keys > lo
    eq = keys == lo
    need = TOPK - jnp.sum(gt.astype(i32), axis=1, keepdims=True)
    col = jax.lax.broadcasted_iota(i32, (BLK, N_ANCHOR), 1)
    e = jnp.where(eq, col, N_ANCHOR * 2)

    lo_j = jnp.full((BLK, 1), -1, i32)
    hi_j = jnp.full((BLK, 1), N_ANCHOR - 1, i32)

    def jbody(_, lohi):
        lo_j, hi_j = lohi
        mid = lo_j + jax.lax.shift_right_logical(hi_j - lo_j, 1)
        cnt = jnp.sum((e <= mid).astype(i32), axis=1, keepdims=True)
        ge = cnt >= need
        return jnp.where(ge, lo_j, mid), jnp.where(ge, mid, hi_j)

    _, hi_j = jax.lax.fori_loop(0, IDX_ITERS, jbody, (lo_j, hi_j), unroll=4)

    mask = gt | (e <= hi_j)
    out_ref[...] = jnp.where(mask, att, 0.0)


@jax.jit
def kernel(context, anchors, W, b):
    b2 = b.reshape(1, 2)
    grid = N_CTX // BLK
    out, cn = pl.pallas_call(
        _agl_kernel,
        grid=(grid,),
        in_specs=[
            pl.BlockSpec((BLK, D_IN), lambda i: (i, 0)),
            pl.BlockSpec((N_ANCHOR, D_IN), lambda i: (0, 0)),
            pl.BlockSpec((D_IN, 2), lambda i: (0, 0)),
            pl.BlockSpec((1, 2), lambda i: (0, 0)),
        ],
        out_specs=[
            pl.BlockSpec((BLK, N_ANCHOR), lambda i: (i, 0)),
            pl.BlockSpec((BLK, 2), lambda i: (i, 0)),
        ],
        out_shape=[
            jax.ShapeDtypeStruct((N_CTX, N_ANCHOR), jnp.float32),
            jax.ShapeDtypeStruct((N_CTX, 2), jnp.float32),
        ],
        scratch_shapes=[pltpu.VMEM((2, N_ANCHOR), jnp.float32),
                        pltpu.VMEM((N_BINS, 2), jnp.float32)],
    )(context, anchors, W, b2)
    return out, cn


# speedup vs baseline: 14.4293x; 1.0696x over previous
import jax
import jax.numpy as jnp
from jax.experimental import pallas as pl
from jax.experimental.pallas import tpu as pltpu

N_CTX = 8192
N_ANCHOR = 1024
D_IN = 1024
TOPK = 64

BLK = 1024
N_BINS = 512
VAL_ITERS = 16
IDX_ITERS = 10
PREP_ITERS = 31


def _kth_largest_key(mat_keys, k, iters):
    i32 = jnp.int32
    rows = mat_keys.shape[0]
    lo = jnp.zeros((rows, 1), i32)
    hi = jnp.max(mat_keys, axis=1, keepdims=True) + 1

    def body(_, lohi):
        lo, hi = lohi
        mid = lo + jax.lax.shift_right_logical(hi - lo, 1)
        cnt = jnp.sum((mat_keys >= mid).astype(i32), axis=1, keepdims=True)
        ge = cnt >= k
        return jnp.where(ge, mid, lo), jnp.where(ge, hi, mid)

    lo, _ = jax.lax.fori_loop(0, iters, body, (lo, hi))
    return lo


def _agl_kernel(ctx_ref, anc_ref, w_ref, b_ref, out_ref, cn_ref, an_ref,
                tab_ref):
    f32 = jnp.float32
    bf16 = jnp.bfloat16
    i32 = jnp.int32

    @pl.when(pl.program_id(0) == 0)
    def _():
        g = jax.lax.dot_general(
            w_ref[...].astype(bf16), anc_ref[...].astype(bf16),
            (((0,), (1,)), ((), ())), preferred_element_type=f32)
        a0 = jnp.maximum(g[0:1, :] + b_ref[0, 0], 0.0)
        a1 = jnp.maximum(g[1:2, :] + b_ref[0, 1], 0.0)
        a0 = a0.astype(bf16).astype(f32)
        a1 = a1.astype(bf16).astype(f32)
        an_ref[0:1, :] = a0
        an_ref[1:2, :] = a1

        d = jax.lax.broadcasted_iota(i32, (N_BINS, 1), 0).astype(f32)
        ta = d * (1.0 / N_BINS)
        tb = (d + 1.0) * (1.0 / N_BINS)
        ga = (1.0 - ta) * a0 + ta * a1
        gb = (1.0 - tb) * a0 + tb * a1
        lo_d = jnp.minimum(ga, gb)
        hi_d = jnp.maximum(ga, gb)
        klo = _kth_largest_key(
            jax.lax.bitcast_convert_type(lo_d, i32), TOPK, PREP_ITERS)
        khi = _kth_largest_key(
            jax.lax.bitcast_convert_type(hi_d, i32), TOPK, PREP_ITERS)
        t_lo = jax.lax.bitcast_convert_type(klo, f32) * (1.0 - 2e-4)
        t_hi = jax.lax.bitcast_convert_type(khi, f32) * (1.0 + 2e-4)
        tab_ref[...] = jnp.concatenate([t_lo, t_hi], axis=1)

    cn = jnp.maximum(
        jax.lax.dot_general(
            ctx_ref[...].astype(bf16), w_ref[...].astype(bf16),
            (((1,), (0,)), ((), ())), preferred_element_type=f32)
        + b_ref[...], 0.0)
    cn_ref[...] = cn

    c0 = cn[:, 0:1].astype(bf16).astype(f32)
    c1 = cn[:, 1:2].astype(bf16).astype(f32)
    att = c0 * an_ref[0:1, :] + c1 * an_ref[1:2, :]

    keys = jax.lax.bitcast_convert_type(att, i32)

    s = c0 + c1
    tau = jnp.where(s > 0.0, c1 / s, 0.0)
    bin_f = jnp.clip(jnp.floor(tau * N_BINS), 0.0, N_BINS - 1.0)
    bin_i = bin_f.astype(i32)
    bins = jax.lax.broadcasted_iota(i32, (BLK, N_BINS), 1)
    onehot = jnp.where(bins == bin_i, 1.0, 0.0).astype(f32)
    t2 = jax.lax.dot_general(
        onehot, tab_ref[...], (((1,), (0,)), ((), ())),
        preferred_element_type=f32,
        precision=jax.lax.Precision.HIGHEST)
    lo_f = jnp.maximum(s * t2[:, 0:1], 0.0)
    hi_f = s * t2[:, 1:2]
    lo = jax.lax.bitcast_convert_type(lo_f, i32)
    hi = jax.lax.bitcast_convert_type(hi_f, i32) + 1
    hi = jnp.maximum(hi, lo + 1)

    def vbody(_, lohi):
        lo, hi = lohi
        mid = lo + jax.lax.shift_right_logical(hi - lo, 1)
        cnt = jnp.sum((keys >= mid).astype(i32), axis=1, keepdims=True)
        ge = cnt >= TOPK
        return jnp.where(ge, mid, lo), jnp.where(ge, hi, mid)

    lo, _ = jax.lax.fori_loop(0, VAL_ITERS, vbody, (lo, hi), unroll=8)

    gt = keys > lo
    eq = keys == lo
    need = TOPK - jnp.sum(gt.astype(i32), axis=1, keepdims=True)
    col = jax.lax.broadcasted_iota(i32, (BLK, N_ANCHOR), 1)
    e = jnp.where(eq, col, N_ANCHOR * 2)

    lo_j = jnp.full((BLK, 1), -1, i32)
    hi_j = jnp.full((BLK, 1), N_ANCHOR - 1, i32)

    def jbody(_, lohi):
        lo_j, hi_j = lohi
        mid = lo_j + jax.lax.shift_right_logical(hi_j - lo_j, 1)
        cnt = jnp.sum((e <= mid).astype(i32), axis=1, keepdims=True)
        ge = cnt >= need
        return jnp.where(ge, lo_j, mid), jnp.where(ge, mid, hi_j)

    _, hi_j = jax.lax.fori_loop(0, IDX_ITERS, jbody, (lo_j, hi_j), unroll=8)

    mask = gt | (e <= hi_j)
    out_ref[...] = jnp.where(mask, att, 0.0)


@jax.jit
def kernel(context, anchors, W, b):
    b2 = b.reshape(1, 2)
    grid = N_CTX // BLK
    out, cn = pl.pallas_call(
        _agl_kernel,
        grid=(grid,),
        in_specs=[
            pl.BlockSpec((BLK, D_IN), lambda i: (i, 0)),
            pl.BlockSpec((N_ANCHOR, D_IN), lambda i: (0, 0)),
            pl.BlockSpec((D_IN, 2), lambda i: (0, 0)),
            pl.BlockSpec((1, 2), lambda i: (0, 0)),
        ],
        out_specs=[
            pl.BlockSpec((BLK, N_ANCHOR), lambda i: (i, 0)),
            pl.BlockSpec((BLK, 2), lambda i: (i, 0)),
        ],
        out_shape=[
            jax.ShapeDtypeStruct((N_CTX, N_ANCHOR), jnp.float32),
            jax.ShapeDtypeStruct((N_CTX, 2), jnp.float32),
        ],
        scratch_shapes=[pltpu.VMEM((2, N_ANCHOR), jnp.float32),
                        pltpu.VMEM((N_BINS, 2), jnp.float32)],
    )(context, anchors, W, b2)
    return out, cn
